# Initial kernel scaffold; baseline (speedup 1.0000x reference)
import jax
import jax.numpy as jnp
from jax.experimental import pallas as pl


def kernel(up_xyz, xyz, up_mask, mask, features):
    raise NotImplementedError("write your pallas kernel here")


# speedup vs baseline: 3.5095x; 3.5095x over previous
import jax
import jax.numpy as jnp
from jax import lax
from jax.experimental import pallas as pl

_QBLK = 512


def _nn_gather_body(q_ref, s_ref, m_ref, f_ref, o_ref):
    q = q_ref[0]
    s = s_ref[0]
    m = m_ref[0]
    f = f_ref[0]
    n = s.shape[1]
    dx = q[:, 0:1] - s[0:1, :]
    dy = q[:, 1:2] - s[1:2, :]
    dz = q[:, 2:3] - s[2:3, :]
    d2 = dx * dx + dy * dy + dz * dz
    d2 = jnp.where(m != 0, d2, jnp.float32(1e10))
    minval = jnp.min(d2, axis=1, keepdims=True)
    iota = lax.broadcasted_iota(jnp.int32, d2.shape, 1)
    idx = jnp.min(jnp.where(d2 == minval, iota, n), axis=1, keepdims=True)
    onehot = (iota == idx).astype(jnp.float32)
    o_ref[0] = lax.dot_general(
        f, onehot, (((1,), (1,)), ((), ())),
        preferred_element_type=jnp.float32,
    )


@jax.jit
def kernel(up_xyz, xyz, up_mask, mask, features):
    del up_mask
    B, C, N = features.shape
    NP = up_xyz.shape[2]
    q_t = jnp.transpose(up_xyz, (0, 2, 1))
    m32 = mask.astype(jnp.int32).reshape(B, 1, N)
    grid = (B, NP // _QBLK)
    return pl.pallas_call(
        _nn_gather_body,
        grid=grid,
        in_specs=[
            pl.BlockSpec((1, _QBLK, 3), lambda b, i: (b, i, 0)),
            pl.BlockSpec((1, 3, N), lambda b, i: (b, 0, 0)),
            pl.BlockSpec((1, 1, N), lambda b, i: (b, 0, 0)),
            pl.BlockSpec((1, C, N), lambda b, i: (b, 0, 0)),
        ],
        out_specs=pl.BlockSpec((1, C, _QBLK), lambda b, i: (b, 0, i)),
        out_shape=jax.ShapeDtypeStruct((B, C, NP), jnp.float32),
    )(q_t, xyz, m32, features)
